# Initial kernel scaffold; baseline (speedup 1.0000x reference)
import jax
import jax.numpy as jnp
from jax.experimental import pallas as pl


def kernel(interest_ids, text_feats, edge_index, emb_table, W_text, b_text, W_gcn1, b_gcn1, W_gcn2, b_gcn2, W_lin3, b_lin3):
    raise NotImplementedError("write your pallas kernel here")


# speedup vs baseline: 4.4316x; 4.4316x over previous
import dataclasses
import functools

import jax
import jax.numpy as jnp
from jax import lax
from jax.experimental import pallas as pl
from jax.experimental.pallas import tpu as pltpu
from jax.experimental.pallas import tpu_sc as plsc

N = 10000
NP = 10240
E = 320000
NC, NS = 2, 16
NW = NC * NS
ECHUNK = 128
NCH = 160
GN = NCH // 8
CCAP = 162
EPAD = NS * NCH * ECHUNK
STRIPE = NP // NS
Q = 2560
ACCR = Q + 128
ASTRIPE = ACCR // NS
ATAIL = Q - (NS - 1) * ASTRIPE
IDCH = 3
IDPAD = NW * IDCH * ECHUNK
BLK = 512
GRID = NP // BLK

_mesh = plsc.VectorSubcoreMesh(core_axis_name="c", subcore_axis_name="s")
f32 = jnp.float32

_sc_params = pltpu.CompilerParams()
if "needs_layout_passes" in pltpu.CompilerParams.__dataclass_fields__:
    _sc_params = dataclasses.replace(_sc_params, needs_layout_passes=False)


@functools.partial(
    pl.kernel,
    out_type=[
        jax.ShapeDtypeStruct((NC, NS, NP // 16, 16), f32),
        jax.ShapeDtypeStruct((IDPAD, 128), f32),
    ],
    mesh=_mesh,
    scratch_types=[
        pltpu.VMEM((NCH, ECHUNK), jnp.int32),
        pltpu.VMEM((IDCH, ECHUNK), jnp.int32),
        pltpu.VMEM((NP // 16, 16), f32),
        pltpu.VMEM((ECHUNK, 128), f32),
    ],
    compiler_params=_sc_params,
)
def _sc_prep(col_hbm, ids_hbm, emb_hbm, zh_hbm,
             hist_hbm, embout_hbm,
             col_v, ids_v, hist_v, gbuf):
    c = lax.axis_index("c")
    s = lax.axis_index("s")
    w = c * NS + s

    pltpu.sync_copy(zh_hbm, hist_v)
    pltpu.sync_copy(col_hbm.at[s], col_v)

    ones = jnp.full((16,), 1.0, f32)

    @pl.loop(c, NCH, step=NC)
    def _(i):
        @pl.loop(0, ECHUNK, step=16)
        def _(k):
            idx = col_v[i, pl.ds(k, 16)]
            plsc.addupdate_scatter(
                hist_v,
                [lax.shift_right_logical(idx, 4), lax.bitwise_and(idx, 15)],
                ones)

    pltpu.sync_copy(hist_v, hist_hbm.at[c, s])

    pltpu.sync_copy(ids_hbm.at[w], ids_v)

    @pl.loop(0, IDCH)
    def _(j):
        pltpu.sync_copy(emb_hbm.at[ids_v.at[j]], gbuf)
        pltpu.sync_copy(gbuf, embout_hbm.at[pl.ds((w * IDCH + j) * ECHUNK,
                                                  ECHUNK)])


@functools.partial(
    pl.kernel,
    out_type=jax.ShapeDtypeStruct((2 * NC, Q, 128), f32),
    mesh=_mesh,
    scratch_types=[
        pltpu.VMEM((8, ECHUNK), jnp.int32),
        pltpu.VMEM((8, ECHUNK), jnp.int32),
        pltpu.VMEM((CCAP, ECHUNK), jnp.int32),
        pltpu.VMEM((CCAP, ECHUNK), jnp.int32),
        pltpu.VMEM((8, 16), jnp.int32),
        pltpu.VMEM((ECHUNK, 128), f32),
        pltpu.VMEM((ASTRIPE, 128), f32),
        pltpu.VMEM_SHARED((ACCR, 128), f32),
    ],
    compiler_params=_sc_params,
)
def _edge_pass(y_hbm, row_hbm, col_hbm, z_hbm, acc_hbm,
               rowg, colg, rowc, colc, cnt_v, gbuf, stage, acc_sp):
    c = lax.axis_index("c")
    s = lax.axis_index("s")

    i0_16 = jnp.zeros((16,), jnp.int32)
    iota16 = lax.iota(jnp.int32, 16)

    for p in range(2):
        base = (c * 2 + p) * Q

        pltpu.sync_copy(z_hbm, stage)
        pltpu.sync_copy(stage, acc_sp.at[pl.ds(s * ASTRIPE, ASTRIPE)])

        cnt_v[0, pl.ds(0, 16)] = i0_16

        @pl.loop(0, GN)
        def _(g):
            pltpu.sync_copy(row_hbm.at[s, pl.ds(g * 8, 8)], rowg)
            pltpu.sync_copy(col_hbm.at[s, pl.ds(g * 8, 8)], colg)

            @pl.loop(0, 8)
            def _(i):
                @pl.loop(0, ECHUNK, step=16)
                def _(k):
                    col = colg[i, pl.ds(k, 16)]
                    row = rowg[i, pl.ds(k, 16)]
                    local = col - base
                    m = jnp.logical_and(local >= 0, local < Q)
                    cnt = cnt_v[0, pl.ds(0, 16)]
                    pos = cnt + plsc.cumsum(m.astype(jnp.int32)) - 1
                    pr = lax.shift_right_logical(pos, 7)
                    pc = lax.bitwise_and(pos, 127)
                    plsc.store_scatter(colc, [pr, pc], local, mask=m)
                    plsc.store_scatter(rowc, [pr, pc], row, mask=m)
                    cnt_v[0, pl.ds(0, 16)] = (
                        cnt + plsc.all_reduce_population_count(m))

        cnt = cnt_v[0, pl.ds(0, 16)]
        limit = jnp.full((16,), CCAP * ECHUNK, jnp.int32)
        for j in range(9):
            idx = cnt + iota16 + (j * 16)
            mok = idx < limit
            ir = lax.shift_right_logical(idx, 7)
            ic = lax.bitwise_and(idx, 127)
            plsc.store_scatter(colc, [ir, ic],
                               jnp.full((16,), Q, jnp.int32), mask=mok)
            plsc.store_scatter(rowc, [ir, ic],
                               jnp.full((16,), NP - 1, jnp.int32), mask=mok)

        trip = lax.shift_right_logical(jnp.max(cnt) + 127, 7)

        plsc.subcore_barrier()

        @pl.loop(0, trip)
        def _(i):
            pltpu.sync_copy(y_hbm.at[rowc.at[i]], gbuf)
            pltpu.sync_copy(gbuf, acc_sp.at[colc.at[i]], add=True)

        plsc.subcore_barrier()

        pltpu.sync_copy(acc_sp.at[pl.ds(s * ASTRIPE, ASTRIPE)], stage)

        @pl.when(s < NS - 1)
        def _():
            pltpu.sync_copy(stage, acc_hbm.at[c * 2 + p,
                                              pl.ds(s * ASTRIPE, ASTRIPE)])

        @pl.when(s == NS - 1)
        def _():
            pltpu.sync_copy(stage.at[pl.ds(0, ATAIL)],
                            acc_hbm.at[c * 2 + p,
                                       pl.ds(s * ASTRIPE, ATAIL)])


def _tc_c_body(emb_ref, text_ref, wt_ref, bt_ref, hist_ref, w1_ref,
               y1_ref, dis_ref):
    t = jnp.dot(text_ref[...], wt_ref[...],
                preferred_element_type=f32) + bt_ref[...]
    x = jnp.concatenate([emb_ref[...][:, :50], t], axis=1)
    xw = jnp.dot(x, w1_ref[...], preferred_element_type=f32)
    deg = jnp.sum(hist_ref[...], axis=0) + 1.0
    dis = lax.rsqrt(deg)
    grow = (pl.program_id(0) * BLK
            + lax.broadcasted_iota(jnp.int32, (BLK, 1), 0))
    y = xw * dis * (grow != NP - 1).astype(f32)
    y1_ref[...] = jnp.concatenate([y, jnp.zeros((BLK, 68), f32)], axis=1)
    dis_ref[...] = dis


def _tc_c(emb, text_p, W_text, b_text, hist, W_gcn1):
    return pl.pallas_call(
        _tc_c_body,
        grid=(GRID,),
        in_specs=[
            pl.BlockSpec((BLK, 128), lambda i: (i, 0)),
            pl.BlockSpec((BLK, 300), lambda i: (i, 0)),
            pl.BlockSpec((300, 80), lambda i: (0, 0)),
            pl.BlockSpec((1, 80), lambda i: (0, 0)),
            pl.BlockSpec((NW, BLK, 1), lambda i: (0, i, 0)),
            pl.BlockSpec((130, 60), lambda i: (0, 0)),
        ],
        out_specs=[
            pl.BlockSpec((BLK, 128), lambda i: (i, 0)),
            pl.BlockSpec((BLK, 1), lambda i: (i, 0)),
        ],
        out_shape=[
            jax.ShapeDtypeStruct((NP, 128), f32),
            jax.ShapeDtypeStruct((NP, 1), f32),
        ],
    )(emb, text_p, W_text, b_text, hist, W_gcn1)


def _tc_e_body(acc_ref, y1_ref, dis_ref, b1_ref, w2_ref, y2_ref):
    a = (acc_ref[0] + y1_ref[...])[:, :60]
    dis = dis_ref[...]
    h = jax.nn.relu(dis * a + b1_ref[...])
    z = jnp.dot(h, w2_ref[...], preferred_element_type=f32)
    grow = (pl.program_id(0) * BLK
            + lax.broadcasted_iota(jnp.int32, (BLK, 1), 0))
    y2 = z * dis * (grow != NP - 1).astype(f32)
    y2_ref[...] = jnp.concatenate([y2, jnp.zeros((BLK, 98), f32)], axis=1)


def _tc_e(acc1, y1, dis, b1, W_gcn2):
    return pl.pallas_call(
        _tc_e_body,
        grid=(GRID,),
        in_specs=[
            pl.BlockSpec((1, BLK, 128), lambda i: (i // 5, i % 5, 0)),
            pl.BlockSpec((BLK, 128), lambda i: (i, 0)),
            pl.BlockSpec((BLK, 1), lambda i: (i, 0)),
            pl.BlockSpec((1, 60), lambda i: (0, 0)),
            pl.BlockSpec((60, 30), lambda i: (0, 0)),
        ],
        out_specs=pl.BlockSpec((BLK, 128), lambda i: (i, 0)),
        out_shape=jax.ShapeDtypeStruct((NP, 128), f32),
    )(acc1, y1, dis, b1, W_gcn2)


def _tc_g_body(acc_ref, y2_ref, dis_ref, b2_ref, w3_ref, b3_ref, out_ref):
    a = (acc_ref[0] + y2_ref[...])[:, :30]
    h = jax.nn.relu(dis_ref[...] * a + b2_ref[...])
    out_ref[...] = jnp.dot(h, w3_ref[...],
                           preferred_element_type=f32) + b3_ref[...]


def _tc_g(acc2, y2, dis, b2, W_lin3, b3):
    return pl.pallas_call(
        _tc_g_body,
        grid=(GRID,),
        in_specs=[
            pl.BlockSpec((1, BLK, 128), lambda i: (i // 5, i % 5, 0)),
            pl.BlockSpec((BLK, 128), lambda i: (i, 0)),
            pl.BlockSpec((BLK, 1), lambda i: (i, 0)),
            pl.BlockSpec((1, 30), lambda i: (0, 0)),
            pl.BlockSpec((30, 18), lambda i: (0, 0)),
            pl.BlockSpec((1, 18), lambda i: (0, 0)),
        ],
        out_specs=pl.BlockSpec((BLK, 18), lambda i: (i, 0)),
        out_shape=jax.ShapeDtypeStruct((NP, 18), f32),
    )(acc2, y2, dis, b2, W_lin3, b3)


def kernel(interest_ids, text_feats, edge_index, emb_table, W_text, b_text,
           W_gcn1, b_gcn1, W_gcn2, b_gcn2, W_lin3, b_lin3):
    i32 = jnp.int32
    fill = jnp.full((EPAD - E,), NP - 1, i32)
    row_p = jnp.concatenate([edge_index[0], fill]).reshape(NS, NCH, ECHUNK)
    col_p = jnp.concatenate([edge_index[1], fill]).reshape(NS, NCH, ECHUNK)
    ids_p = jnp.concatenate(
        [interest_ids.astype(i32), jnp.zeros((IDPAD - N,), i32)]
    ).reshape(NW, IDCH, ECHUNK)
    emb_pad = jnp.pad(emb_table, ((0, 0), (0, 78)))
    text_p = jnp.pad(text_feats, ((0, NP - N), (0, 0)))
    zh = jnp.zeros((NP // 16, 16), f32)
    z128 = jnp.zeros((ASTRIPE, 128), f32)

    hist, emb = _sc_prep(col_p, ids_p, emb_pad, zh)
    hist = hist.reshape(NW, NP, 1)

    y1, dis = _tc_c(emb[:NP], text_p, W_text, b_text.reshape(1, 80), hist,
                    W_gcn1)
    acc1 = _edge_pass(y1, row_p, col_p, z128)
    y2 = _tc_e(acc1, y1, dis, b_gcn1.reshape(1, 60), W_gcn2)
    acc2 = _edge_pass(y2, row_p, col_p, z128)
    out = _tc_g(acc2, y2, dis, b_gcn2.reshape(1, 30), W_lin3,
                b_lin3.reshape(1, 18))
    return out[:N]


# speedup vs baseline: 4.4356x; 1.0009x over previous
import dataclasses
import functools

import jax
import jax.numpy as jnp
from jax import lax
from jax.experimental import pallas as pl
from jax.experimental.pallas import tpu as pltpu
from jax.experimental.pallas import tpu_sc as plsc

N = 10000
NP = 10240
E = 320000
NC, NS = 2, 16
NW = NC * NS
ECHUNK = 128
NCH = 160
GN = NCH // 8
CCAP = 162
EPAD = NS * NCH * ECHUNK
STRIPE = NP // NS
Q = 2560
ACCR = Q + 128
ASTRIPE = ACCR // NS
ATAIL = Q - (NS - 1) * ASTRIPE
IDCH = 3
IDPAD = NW * IDCH * ECHUNK
BLK = 512
GRID = NP // BLK

_mesh = plsc.VectorSubcoreMesh(core_axis_name="c", subcore_axis_name="s")
f32 = jnp.float32

_sc_params = pltpu.CompilerParams()
if "needs_layout_passes" in pltpu.CompilerParams.__dataclass_fields__:
    _sc_params = dataclasses.replace(_sc_params, needs_layout_passes=False)


@functools.partial(
    pl.kernel,
    out_type=[
        jax.ShapeDtypeStruct((NC, NS, NP // 16, 16), f32),
        jax.ShapeDtypeStruct((IDPAD, 128), f32),
    ],
    mesh=_mesh,
    scratch_types=[
        pltpu.VMEM((NCH, ECHUNK), jnp.int32),
        pltpu.VMEM((IDCH, ECHUNK), jnp.int32),
        pltpu.VMEM((NP // 16, 16), f32),
        pltpu.VMEM((ECHUNK, 128), f32),
    ],
    compiler_params=_sc_params,
)
def _sc_prep(col_hbm, ids_hbm, emb_hbm, zh_hbm,
             hist_hbm, embout_hbm,
             col_v, ids_v, hist_v, gbuf):
    c = lax.axis_index("c")
    s = lax.axis_index("s")
    w = c * NS + s

    pltpu.sync_copy(zh_hbm, hist_v)
    pltpu.sync_copy(col_hbm.at[s], col_v)

    ones = jnp.full((16,), 1.0, f32)

    @pl.loop(c, NCH, step=NC)
    def _(i):
        @pl.loop(0, ECHUNK, step=16)
        def _(k):
            idx = col_v[i, pl.ds(k, 16)]
            plsc.addupdate_scatter(
                hist_v,
                [lax.shift_right_logical(idx, 4), lax.bitwise_and(idx, 15)],
                ones)

    pltpu.sync_copy(hist_v, hist_hbm.at[c, s])

    pltpu.sync_copy(ids_hbm.at[w], ids_v)

    @pl.loop(0, IDCH)
    def _(j):
        pltpu.sync_copy(emb_hbm.at[ids_v.at[j]], gbuf)
        pltpu.sync_copy(gbuf, embout_hbm.at[pl.ds((w * IDCH + j) * ECHUNK,
                                                  ECHUNK)])


@functools.partial(
    pl.kernel,
    out_type=jax.ShapeDtypeStruct((2 * NC, Q, 128), f32),
    mesh=_mesh,
    scratch_types=[
        pltpu.VMEM((8, ECHUNK), jnp.int32),
        pltpu.VMEM((8, ECHUNK), jnp.int32),
        pltpu.VMEM((CCAP, ECHUNK), jnp.int32),
        pltpu.VMEM((CCAP, ECHUNK), jnp.int32),
        pltpu.VMEM((8, 16), jnp.int32),
        pltpu.VMEM((ECHUNK, 128), f32),
        pltpu.VMEM((ASTRIPE, 128), f32),
        pltpu.VMEM_SHARED((ACCR, 128), f32),
    ],
    compiler_params=_sc_params,
)
def _edge_pass(y_hbm, row_hbm, col_hbm, z_hbm, acc_hbm,
               rowg, colg, rowc, colc, cnt_v, gbuf, stage, acc_sp):
    c = lax.axis_index("c")
    s = lax.axis_index("s")

    i0_16 = jnp.zeros((16,), jnp.int32)
    iota16 = lax.iota(jnp.int32, 16)

    for p in range(2):
        base = (c * 2 + p) * Q

        pltpu.sync_copy(z_hbm, stage)
        pltpu.sync_copy(stage, acc_sp.at[pl.ds(s * ASTRIPE, ASTRIPE)])

        cnt_v[0, pl.ds(0, 16)] = i0_16

        @pl.loop(0, GN)
        def _(g):
            pltpu.sync_copy(row_hbm.at[s, pl.ds(g * 8, 8)], rowg)
            pltpu.sync_copy(col_hbm.at[s, pl.ds(g * 8, 8)], colg)

            cnt = cnt_v[0, pl.ds(0, 16)]
            for i in range(8):
                for k in range(0, ECHUNK, 16):
                    col = colg[i, pl.ds(k, 16)]
                    row = rowg[i, pl.ds(k, 16)]
                    local = col - base
                    m = jnp.logical_and(local >= 0, local < Q)
                    pos = cnt + plsc.cumsum(m.astype(jnp.int32)) - 1
                    pr = lax.shift_right_logical(pos, 7)
                    pc = lax.bitwise_and(pos, 127)
                    plsc.store_scatter(colc, [pr, pc], local, mask=m)
                    plsc.store_scatter(rowc, [pr, pc], row, mask=m)
                    cnt = cnt + plsc.all_reduce_population_count(m)
            cnt_v[0, pl.ds(0, 16)] = cnt

        cnt = cnt_v[0, pl.ds(0, 16)]
        limit = jnp.full((16,), CCAP * ECHUNK, jnp.int32)
        for j in range(9):
            idx = cnt + iota16 + (j * 16)
            mok = idx < limit
            ir = lax.shift_right_logical(idx, 7)
            ic = lax.bitwise_and(idx, 127)
            plsc.store_scatter(colc, [ir, ic],
                               jnp.full((16,), Q, jnp.int32), mask=mok)
            plsc.store_scatter(rowc, [ir, ic],
                               jnp.full((16,), NP - 1, jnp.int32), mask=mok)

        trip = lax.shift_right_logical(jnp.max(cnt) + 127, 7)

        plsc.subcore_barrier()

        @pl.loop(0, trip)
        def _(i):
            pltpu.sync_copy(y_hbm.at[rowc.at[i]], gbuf)
            pltpu.sync_copy(gbuf, acc_sp.at[colc.at[i]], add=True)

        plsc.subcore_barrier()

        pltpu.sync_copy(acc_sp.at[pl.ds(s * ASTRIPE, ASTRIPE)], stage)

        @pl.when(s < NS - 1)
        def _():
            pltpu.sync_copy(stage, acc_hbm.at[c * 2 + p,
                                              pl.ds(s * ASTRIPE, ASTRIPE)])

        @pl.when(s == NS - 1)
        def _():
            pltpu.sync_copy(stage.at[pl.ds(0, ATAIL)],
                            acc_hbm.at[c * 2 + p,
                                       pl.ds(s * ASTRIPE, ATAIL)])


def _tc_c_body(emb_ref, text_ref, wt_ref, bt_ref, hist_ref, w1_ref,
               y1_ref, dis_ref):
    t = jnp.dot(text_ref[...], wt_ref[...],
                preferred_element_type=f32) + bt_ref[...]
    x = jnp.concatenate([emb_ref[...][:, :50], t], axis=1)
    xw = jnp.dot(x, w1_ref[...], preferred_element_type=f32)
    deg = jnp.sum(hist_ref[...], axis=0) + 1.0
    dis = lax.rsqrt(deg)
    grow = (pl.program_id(0) * BLK
            + lax.broadcasted_iota(jnp.int32, (BLK, 1), 0))
    y = xw * dis * (grow != NP - 1).astype(f32)
    y1_ref[...] = jnp.concatenate([y, jnp.zeros((BLK, 68), f32)], axis=1)
    dis_ref[...] = dis


def _tc_c(emb, text_p, W_text, b_text, hist, W_gcn1):
    return pl.pallas_call(
        _tc_c_body,
        grid=(GRID,),
        in_specs=[
            pl.BlockSpec((BLK, 128), lambda i: (i, 0)),
            pl.BlockSpec((BLK, 300), lambda i: (i, 0)),
            pl.BlockSpec((300, 80), lambda i: (0, 0)),
            pl.BlockSpec((1, 80), lambda i: (0, 0)),
            pl.BlockSpec((NW, BLK, 1), lambda i: (0, i, 0)),
            pl.BlockSpec((130, 60), lambda i: (0, 0)),
        ],
        out_specs=[
            pl.BlockSpec((BLK, 128), lambda i: (i, 0)),
            pl.BlockSpec((BLK, 1), lambda i: (i, 0)),
        ],
        out_shape=[
            jax.ShapeDtypeStruct((NP, 128), f32),
            jax.ShapeDtypeStruct((NP, 1), f32),
        ],
    )(emb, text_p, W_text, b_text, hist, W_gcn1)


def _tc_e_body(acc_ref, y1_ref, dis_ref, b1_ref, w2_ref, y2_ref):
    a = (acc_ref[0] + y1_ref[...])[:, :60]
    dis = dis_ref[...]
    h = jax.nn.relu(dis * a + b1_ref[...])
    z = jnp.dot(h, w2_ref[...], preferred_element_type=f32)
    grow = (pl.program_id(0) * BLK
            + lax.broadcasted_iota(jnp.int32, (BLK, 1), 0))
    y2 = z * dis * (grow != NP - 1).astype(f32)
    y2_ref[...] = jnp.concatenate([y2, jnp.zeros((BLK, 98), f32)], axis=1)


def _tc_e(acc1, y1, dis, b1, W_gcn2):
    return pl.pallas_call(
        _tc_e_body,
        grid=(GRID,),
        in_specs=[
            pl.BlockSpec((1, BLK, 128), lambda i: (i // 5, i % 5, 0)),
            pl.BlockSpec((BLK, 128), lambda i: (i, 0)),
            pl.BlockSpec((BLK, 1), lambda i: (i, 0)),
            pl.BlockSpec((1, 60), lambda i: (0, 0)),
            pl.BlockSpec((60, 30), lambda i: (0, 0)),
        ],
        out_specs=pl.BlockSpec((BLK, 128), lambda i: (i, 0)),
        out_shape=jax.ShapeDtypeStruct((NP, 128), f32),
    )(acc1, y1, dis, b1, W_gcn2)


def _tc_g_body(acc_ref, y2_ref, dis_ref, b2_ref, w3_ref, b3_ref, out_ref):
    a = (acc_ref[0] + y2_ref[...])[:, :30]
    h = jax.nn.relu(dis_ref[...] * a + b2_ref[...])
    out_ref[...] = jnp.dot(h, w3_ref[...],
                           preferred_element_type=f32) + b3_ref[...]


def _tc_g(acc2, y2, dis, b2, W_lin3, b3):
    return pl.pallas_call(
        _tc_g_body,
        grid=(GRID,),
        in_specs=[
            pl.BlockSpec((1, BLK, 128), lambda i: (i // 5, i % 5, 0)),
            pl.BlockSpec((BLK, 128), lambda i: (i, 0)),
            pl.BlockSpec((BLK, 1), lambda i: (i, 0)),
            pl.BlockSpec((1, 30), lambda i: (0, 0)),
            pl.BlockSpec((30, 18), lambda i: (0, 0)),
            pl.BlockSpec((1, 18), lambda i: (0, 0)),
        ],
        out_specs=pl.BlockSpec((BLK, 18), lambda i: (i, 0)),
        out_shape=jax.ShapeDtypeStruct((NP, 18), f32),
    )(acc2, y2, dis, b2, W_lin3, b3)


def kernel(interest_ids, text_feats, edge_index, emb_table, W_text, b_text,
           W_gcn1, b_gcn1, W_gcn2, b_gcn2, W_lin3, b_lin3):
    i32 = jnp.int32
    fill = jnp.full((EPAD - E,), NP - 1, i32)
    row_p = jnp.concatenate([edge_index[0], fill]).reshape(NS, NCH, ECHUNK)
    col_p = jnp.concatenate([edge_index[1], fill]).reshape(NS, NCH, ECHUNK)
    ids_p = jnp.concatenate(
        [interest_ids.astype(i32), jnp.zeros((IDPAD - N,), i32)]
    ).reshape(NW, IDCH, ECHUNK)
    emb_pad = jnp.pad(emb_table, ((0, 0), (0, 78)))
    text_p = jnp.pad(text_feats, ((0, NP - N), (0, 0)))
    zh = jnp.zeros((NP // 16, 16), f32)
    z128 = jnp.zeros((ASTRIPE, 128), f32)

    hist, emb = _sc_prep(col_p, ids_p, emb_pad, zh)
    hist = hist.reshape(NW, NP, 1)

    y1, dis = _tc_c(emb[:NP], text_p, W_text, b_text.reshape(1, 80), hist,
                    W_gcn1)
    acc1 = _edge_pass(y1, row_p, col_p, z128)
    y2 = _tc_e(acc1, y1, dis, b_gcn1.reshape(1, 60), W_gcn2)
    acc2 = _edge_pass(y2, row_p, col_p, z128)
    out = _tc_g(acc2, y2, dis, b_gcn2.reshape(1, 30), W_lin3,
                b_lin3.reshape(1, 18))
    return out[:N]


# speedup vs baseline: 4.8006x; 1.0823x over previous
import dataclasses
import functools

import jax
import jax.numpy as jnp
from jax import lax
from jax.experimental import pallas as pl
from jax.experimental.pallas import tpu as pltpu
from jax.experimental.pallas import tpu_sc as plsc

N = 10000
NP = 10240
E = 320000
NC, NS = 2, 16
NW = NC * NS
ECHUNK = 128
NCH = 160
GN = NCH // 8
CCAP = 162
EPAD = NS * NCH * ECHUNK
STRIPE = NP // NS
PHASES = (((0, 3584), (3584, 3072)), ((6656, 1536), (8192, 2048)))
ACCR = 3712
ASTRIPE = ACCR // NS
JROW = 3584
IDCH = 3
IDPAD = NW * IDCH * ECHUNK
BLK = 512
GRID = NP // BLK

_mesh = plsc.VectorSubcoreMesh(core_axis_name="c", subcore_axis_name="s")
f32 = jnp.float32

_sc_params = pltpu.CompilerParams()
if "needs_layout_passes" in pltpu.CompilerParams.__dataclass_fields__:
    _sc_params = dataclasses.replace(_sc_params, needs_layout_passes=False)


@functools.partial(
    pl.kernel,
    out_type=[
        jax.ShapeDtypeStruct((NC, NS, NP // 16, 16), f32),
        jax.ShapeDtypeStruct((IDPAD, 128), f32),
    ],
    mesh=_mesh,
    scratch_types=[
        pltpu.VMEM((NCH, ECHUNK), jnp.int32),
        pltpu.VMEM((IDCH, ECHUNK), jnp.int32),
        pltpu.VMEM((NP // 16, 16), f32),
        pltpu.VMEM((ECHUNK, 128), f32),
    ],
    compiler_params=_sc_params,
)
def _sc_prep(col_hbm, ids_hbm, emb_hbm, zh_hbm,
             hist_hbm, embout_hbm,
             col_v, ids_v, hist_v, gbuf):
    c = lax.axis_index("c")
    s = lax.axis_index("s")
    w = c * NS + s

    pltpu.sync_copy(zh_hbm, hist_v)
    pltpu.sync_copy(col_hbm.at[s], col_v)

    ones = jnp.full((16,), 1.0, f32)

    @pl.loop(c, NCH, step=NC)
    def _(i):
        @pl.loop(0, ECHUNK, step=16)
        def _(k):
            idx = col_v[i, pl.ds(k, 16)]
            plsc.addupdate_scatter(
                hist_v,
                [lax.shift_right_logical(idx, 4), lax.bitwise_and(idx, 15)],
                ones)

    pltpu.sync_copy(hist_v, hist_hbm.at[c, s])

    pltpu.sync_copy(ids_hbm.at[w], ids_v)

    @pl.loop(0, IDCH)
    def _(j):
        pltpu.sync_copy(emb_hbm.at[ids_v.at[j]], gbuf)
        pltpu.sync_copy(gbuf, embout_hbm.at[pl.ds((w * IDCH + j) * ECHUNK,
                                                  ECHUNK)])


@functools.partial(
    pl.kernel,
    out_type=jax.ShapeDtypeStruct((NP, 128), f32),
    mesh=_mesh,
    scratch_types=[
        pltpu.VMEM((8, ECHUNK), jnp.int32),
        pltpu.VMEM((8, ECHUNK), jnp.int32),
        pltpu.VMEM((CCAP, ECHUNK), jnp.int32),
        pltpu.VMEM((CCAP, ECHUNK), jnp.int32),
        pltpu.VMEM((8, 16), jnp.int32),
        pltpu.VMEM((ECHUNK, 128), f32),
        pltpu.VMEM((ASTRIPE, 128), f32),
        pltpu.VMEM_SHARED((ACCR, 128), f32),
    ],
    compiler_params=_sc_params,
)
def _edge_pass(y_hbm, row_hbm, col_hbm, z_hbm, acc_hbm,
               rowg, colg, rowc, colc, cnt_v, gbuf, stage, acc_sp):
    c = lax.axis_index("c")
    s = lax.axis_index("s")

    i0_16 = jnp.zeros((16,), jnp.int32)
    iota16 = lax.iota(jnp.int32, 16)

    def do_phase(gstart, size):
        pltpu.sync_copy(z_hbm, stage)
        pltpu.sync_copy(stage, acc_sp.at[pl.ds(s * ASTRIPE, ASTRIPE)])

        cnt_v[0, pl.ds(0, 16)] = i0_16

        @pl.loop(0, GN)
        def _(g):
            pltpu.sync_copy(row_hbm.at[s, pl.ds(g * 8, 8)], rowg)
            pltpu.sync_copy(col_hbm.at[s, pl.ds(g * 8, 8)], colg)

            cnt = cnt_v[0, pl.ds(0, 16)]
            for i in range(8):
                for k in range(0, ECHUNK, 16):
                    col = colg[i, pl.ds(k, 16)]
                    row = rowg[i, pl.ds(k, 16)]
                    local = col - gstart
                    m = jnp.logical_and(local >= 0, local < size)
                    pos = cnt + plsc.cumsum(m.astype(jnp.int32)) - 1
                    pr = lax.shift_right_logical(pos, 7)
                    pc = lax.bitwise_and(pos, 127)
                    plsc.store_scatter(colc, [pr, pc], local, mask=m)
                    plsc.store_scatter(rowc, [pr, pc], row, mask=m)
                    cnt = cnt + plsc.all_reduce_population_count(m)
            cnt_v[0, pl.ds(0, 16)] = cnt

        cnt = cnt_v[0, pl.ds(0, 16)]
        limit = jnp.full((16,), CCAP * ECHUNK, jnp.int32)
        for j in range(9):
            idx = cnt + iota16 + (j * 16)
            mok = idx < limit
            ir = lax.shift_right_logical(idx, 7)
            ic = lax.bitwise_and(idx, 127)
            plsc.store_scatter(colc, [ir, ic],
                               jnp.full((16,), JROW, jnp.int32), mask=mok)
            plsc.store_scatter(rowc, [ir, ic],
                               jnp.full((16,), NP - 1, jnp.int32), mask=mok)

        trip = lax.shift_right_logical(jnp.max(cnt) + 127, 7)

        plsc.subcore_barrier()

        @pl.loop(0, trip)
        def _(i):
            pltpu.sync_copy(y_hbm.at[rowc.at[i]], gbuf)
            pltpu.sync_copy(gbuf, acc_sp.at[colc.at[i]], add=True)

        plsc.subcore_barrier()

        sz16 = size // NS
        pltpu.sync_copy(acc_sp.at[pl.ds(s * sz16, sz16)],
                        stage.at[pl.ds(0, sz16)])
        pltpu.sync_copy(stage.at[pl.ds(0, sz16)],
                        acc_hbm.at[pl.ds(gstart + s * sz16, sz16)])

    for ci in range(NC):
        @pl.when(c == ci)
        def _():
            for p in range(2):
                do_phase(*PHASES[ci][p])


def _tc_c_body(emb_ref, text_ref, wt_ref, bt_ref, hist_ref, w1_ref,
               y1_ref, dis_ref):
    t = jnp.dot(text_ref[...], wt_ref[...],
                preferred_element_type=f32) + bt_ref[...]
    x = jnp.concatenate([emb_ref[...][:, :50], t], axis=1)
    xw = jnp.dot(x, w1_ref[...], preferred_element_type=f32)
    deg = jnp.sum(hist_ref[...], axis=0) + 1.0
    dis = lax.rsqrt(deg)
    grow = (pl.program_id(0) * BLK
            + lax.broadcasted_iota(jnp.int32, (BLK, 1), 0))
    y = xw * dis * (grow != NP - 1).astype(f32)
    y1_ref[...] = jnp.concatenate([y, jnp.zeros((BLK, 68), f32)], axis=1)
    dis_ref[...] = dis


def _tc_c(emb, text_p, W_text, b_text, hist, W_gcn1):
    return pl.pallas_call(
        _tc_c_body,
        grid=(GRID,),
        in_specs=[
            pl.BlockSpec((BLK, 128), lambda i: (i, 0)),
            pl.BlockSpec((BLK, 300), lambda i: (i, 0)),
            pl.BlockSpec((300, 80), lambda i: (0, 0)),
            pl.BlockSpec((1, 80), lambda i: (0, 0)),
            pl.BlockSpec((NW, BLK, 1), lambda i: (0, i, 0)),
            pl.BlockSpec((130, 60), lambda i: (0, 0)),
        ],
        out_specs=[
            pl.BlockSpec((BLK, 128), lambda i: (i, 0)),
            pl.BlockSpec((BLK, 1), lambda i: (i, 0)),
        ],
        out_shape=[
            jax.ShapeDtypeStruct((NP, 128), f32),
            jax.ShapeDtypeStruct((NP, 1), f32),
        ],
    )(emb, text_p, W_text, b_text, hist, W_gcn1)


def _tc_e_body(acc_ref, y1_ref, dis_ref, b1_ref, w2_ref, y2_ref):
    a = (acc_ref[...] + y1_ref[...])[:, :60]
    dis = dis_ref[...]
    h = jax.nn.relu(dis * a + b1_ref[...])
    z = jnp.dot(h, w2_ref[...], preferred_element_type=f32)
    grow = (pl.program_id(0) * BLK
            + lax.broadcasted_iota(jnp.int32, (BLK, 1), 0))
    y2 = z * dis * (grow != NP - 1).astype(f32)
    y2_ref[...] = jnp.concatenate([y2, jnp.zeros((BLK, 98), f32)], axis=1)


def _tc_e(acc1, y1, dis, b1, W_gcn2):
    return pl.pallas_call(
        _tc_e_body,
        grid=(GRID,),
        in_specs=[
            pl.BlockSpec((BLK, 128), lambda i: (i, 0)),
            pl.BlockSpec((BLK, 128), lambda i: (i, 0)),
            pl.BlockSpec((BLK, 1), lambda i: (i, 0)),
            pl.BlockSpec((1, 60), lambda i: (0, 0)),
            pl.BlockSpec((60, 30), lambda i: (0, 0)),
        ],
        out_specs=pl.BlockSpec((BLK, 128), lambda i: (i, 0)),
        out_shape=jax.ShapeDtypeStruct((NP, 128), f32),
    )(acc1, y1, dis, b1, W_gcn2)


def _tc_g_body(acc_ref, y2_ref, dis_ref, b2_ref, w3_ref, b3_ref, out_ref):
    a = (acc_ref[...] + y2_ref[...])[:, :30]
    h = jax.nn.relu(dis_ref[...] * a + b2_ref[...])
    out_ref[...] = jnp.dot(h, w3_ref[...],
                           preferred_element_type=f32) + b3_ref[...]


def _tc_g(acc2, y2, dis, b2, W_lin3, b3):
    return pl.pallas_call(
        _tc_g_body,
        grid=(GRID,),
        in_specs=[
            pl.BlockSpec((BLK, 128), lambda i: (i, 0)),
            pl.BlockSpec((BLK, 128), lambda i: (i, 0)),
            pl.BlockSpec((BLK, 1), lambda i: (i, 0)),
            pl.BlockSpec((1, 30), lambda i: (0, 0)),
            pl.BlockSpec((30, 18), lambda i: (0, 0)),
            pl.BlockSpec((1, 18), lambda i: (0, 0)),
        ],
        out_specs=pl.BlockSpec((BLK, 18), lambda i: (i, 0)),
        out_shape=jax.ShapeDtypeStruct((NP, 18), f32),
    )(acc2, y2, dis, b2, W_lin3, b3)


def kernel(interest_ids, text_feats, edge_index, emb_table, W_text, b_text,
           W_gcn1, b_gcn1, W_gcn2, b_gcn2, W_lin3, b_lin3):
    i32 = jnp.int32
    fill = jnp.full((EPAD - E,), NP - 1, i32)
    row_p = jnp.concatenate([edge_index[0], fill]).reshape(NS, NCH, ECHUNK)
    col_p = jnp.concatenate([edge_index[1], fill]).reshape(NS, NCH, ECHUNK)
    ids_p = jnp.concatenate(
        [interest_ids.astype(i32), jnp.zeros((IDPAD - N,), i32)]
    ).reshape(NW, IDCH, ECHUNK)
    emb_pad = jnp.pad(emb_table, ((0, 0), (0, 78)))
    text_p = jnp.pad(text_feats, ((0, NP - N), (0, 0)))
    zh = jnp.zeros((NP // 16, 16), f32)
    z128 = jnp.zeros((ASTRIPE, 128), f32)

    hist, emb = _sc_prep(col_p, ids_p, emb_pad, zh)
    hist = hist.reshape(NW, NP, 1)

    y1, dis = _tc_c(emb[:NP], text_p, W_text, b_text.reshape(1, 80), hist,
                    W_gcn1)
    acc1 = _edge_pass(y1, row_p, col_p, z128)
    y2 = _tc_e(acc1, y1, dis, b_gcn1.reshape(1, 60), W_gcn2)
    acc2 = _edge_pass(y2, row_p, col_p, z128)
    out = _tc_g(acc2, y2, dis, b_gcn2.reshape(1, 30), W_lin3,
                b_lin3.reshape(1, 18))
    return out[:N]


# speedup vs baseline: 4.9388x; 1.0288x over previous
import dataclasses
import functools

import jax
import jax.numpy as jnp
from jax import lax
from jax.experimental import pallas as pl
from jax.experimental.pallas import tpu as pltpu
from jax.experimental.pallas import tpu_sc as plsc

N = 10000
NP = 10240
E = 320000
NC, NS = 2, 16
NW = NC * NS
ECHUNK = 128
NCH = 160
GN = NCH // 8
CCAP = 162
EPAD = NS * NCH * ECHUNK
STRIPE = NP // NS
PHASES = (((6656, 1536), (8192, 2048)), ((0, 3584), (3584, 3072)))
ACCR = 3712
ASTRIPE = ACCR // NS
JROW = 3584
IDCH = 3
IDPAD = NW * IDCH * ECHUNK
BLK = 512
GRID = NP // BLK

_mesh = plsc.VectorSubcoreMesh(core_axis_name="c", subcore_axis_name="s")
f32 = jnp.float32

_sc_params = pltpu.CompilerParams()
if "needs_layout_passes" in pltpu.CompilerParams.__dataclass_fields__:
    _sc_params = dataclasses.replace(_sc_params, needs_layout_passes=False)


@functools.partial(
    pl.kernel,
    out_type=[
        jax.ShapeDtypeStruct((NC, NS, NP // 16, 16), f32),
        jax.ShapeDtypeStruct((IDPAD, 128), f32),
    ],
    mesh=_mesh,
    scratch_types=[
        pltpu.VMEM((NCH, ECHUNK), jnp.int32),
        pltpu.VMEM((IDCH, ECHUNK), jnp.int32),
        pltpu.VMEM((NP // 16, 16), f32),
        pltpu.VMEM((ECHUNK, 128), f32),
    ],
    compiler_params=_sc_params,
)
def _sc_prep(col_hbm, ids_hbm, emb_hbm, zh_hbm,
             hist_hbm, embout_hbm,
             col_v, ids_v, hist_v, gbuf):
    c = lax.axis_index("c")
    s = lax.axis_index("s")
    w = c * NS + s

    pltpu.sync_copy(zh_hbm, hist_v)
    pltpu.sync_copy(col_hbm.at[s], col_v)

    ones = jnp.full((16,), 1.0, f32)

    @pl.loop(c, NCH, step=NC)
    def _(i):
        @pl.loop(0, ECHUNK, step=16)
        def _(k):
            idx = col_v[i, pl.ds(k, 16)]
            plsc.addupdate_scatter(
                hist_v,
                [lax.shift_right_logical(idx, 4), lax.bitwise_and(idx, 15)],
                ones)

    pltpu.sync_copy(hist_v, hist_hbm.at[c, s])

    pltpu.sync_copy(ids_hbm.at[w], ids_v)

    @pl.loop(0, IDCH)
    def _(j):
        pltpu.sync_copy(emb_hbm.at[ids_v.at[j]], gbuf)
        pltpu.sync_copy(gbuf, embout_hbm.at[pl.ds((w * IDCH + j) * ECHUNK,
                                                  ECHUNK)])


@functools.partial(
    pl.kernel,
    out_type=jax.ShapeDtypeStruct((NP, 128), f32),
    mesh=_mesh,
    scratch_types=[
        pltpu.VMEM((8, ECHUNK), jnp.int32),
        pltpu.VMEM((8, ECHUNK), jnp.int32),
        pltpu.VMEM((CCAP, ECHUNK), jnp.int32),
        pltpu.VMEM((CCAP, ECHUNK), jnp.int32),
        pltpu.VMEM((8, 16), jnp.int32),
        pltpu.VMEM((ECHUNK, 128), f32),
        pltpu.VMEM((ASTRIPE, 128), f32),
        pltpu.VMEM_SHARED((ACCR, 128), f32),
    ],
    compiler_params=_sc_params,
)
def _edge_pass(y_hbm, row_hbm, col_hbm, z_hbm, acc_hbm,
               rowg, colg, rowc, colc, cnt_v, gbuf, stage, acc_sp):
    c = lax.axis_index("c")
    s = lax.axis_index("s")

    i0_16 = jnp.zeros((16,), jnp.int32)
    iota16 = lax.iota(jnp.int32, 16)

    def do_phase(gstart, size):
        pltpu.sync_copy(z_hbm, stage)
        pltpu.sync_copy(stage, acc_sp.at[pl.ds(s * ASTRIPE, ASTRIPE)])

        cnt_v[0, pl.ds(0, 16)] = i0_16

        @pl.loop(0, GN)
        def _(g):
            pltpu.sync_copy(row_hbm.at[s, pl.ds(g * 8, 8)], rowg)
            pltpu.sync_copy(col_hbm.at[s, pl.ds(g * 8, 8)], colg)

            cnt = cnt_v[0, pl.ds(0, 16)]
            for i in range(8):
                for k in range(0, ECHUNK, 16):
                    col = colg[i, pl.ds(k, 16)]
                    row = rowg[i, pl.ds(k, 16)]
                    local = col - gstart
                    m = jnp.logical_and(local >= 0, local < size)
                    pos = cnt + plsc.cumsum(m.astype(jnp.int32)) - 1
                    pr = lax.shift_right_logical(pos, 7)
                    pc = lax.bitwise_and(pos, 127)
                    plsc.store_scatter(colc, [pr, pc], local, mask=m)
                    plsc.store_scatter(rowc, [pr, pc], row, mask=m)
                    cnt = cnt + plsc.all_reduce_population_count(m)
            cnt_v[0, pl.ds(0, 16)] = cnt

        cnt = cnt_v[0, pl.ds(0, 16)]
        limit = jnp.full((16,), CCAP * ECHUNK, jnp.int32)
        for j in range(9):
            idx = cnt + iota16 + (j * 16)
            mok = idx < limit
            ir = lax.shift_right_logical(idx, 7)
            ic = lax.bitwise_and(idx, 127)
            plsc.store_scatter(colc, [ir, ic],
                               jnp.full((16,), JROW, jnp.int32), mask=mok)
            plsc.store_scatter(rowc, [ir, ic],
                               jnp.full((16,), NP - 1, jnp.int32), mask=mok)

        trip = lax.shift_right_logical(jnp.max(cnt) + 127, 7)

        plsc.subcore_barrier()

        @pl.loop(0, trip)
        def _(i):
            pltpu.sync_copy(y_hbm.at[rowc.at[i]], gbuf)
            pltpu.sync_copy(gbuf, acc_sp.at[colc.at[i]], add=True)

        plsc.subcore_barrier()

        sz16 = size // NS
        pltpu.sync_copy(acc_sp.at[pl.ds(s * sz16, sz16)],
                        stage.at[pl.ds(0, sz16)])
        pltpu.sync_copy(stage.at[pl.ds(0, sz16)],
                        acc_hbm.at[pl.ds(gstart + s * sz16, sz16)])

    for ci in range(NC):
        @pl.when(c == ci)
        def _():
            for p in range(2):
                do_phase(*PHASES[ci][p])


def _tc_c_body(emb_ref, text_ref, wt_ref, bt_ref, hist_ref, w1_ref,
               y1_ref, dis_ref):
    t = jnp.dot(text_ref[...], wt_ref[...],
                preferred_element_type=f32) + bt_ref[...]
    x = jnp.concatenate([emb_ref[...][:, :50], t], axis=1)
    xw = jnp.dot(x, w1_ref[...], preferred_element_type=f32)
    deg = jnp.sum(hist_ref[...], axis=0) + 1.0
    dis = lax.rsqrt(deg)
    grow = (pl.program_id(0) * BLK
            + lax.broadcasted_iota(jnp.int32, (BLK, 1), 0))
    y = xw * dis * (grow != NP - 1).astype(f32)
    y1_ref[...] = jnp.concatenate([y, jnp.zeros((BLK, 68), f32)], axis=1)
    dis_ref[...] = dis


def _tc_c(emb, text_p, W_text, b_text, hist, W_gcn1):
    return pl.pallas_call(
        _tc_c_body,
        grid=(GRID,),
        in_specs=[
            pl.BlockSpec((BLK, 128), lambda i: (i, 0)),
            pl.BlockSpec((BLK, 300), lambda i: (i, 0)),
            pl.BlockSpec((300, 80), lambda i: (0, 0)),
            pl.BlockSpec((1, 80), lambda i: (0, 0)),
            pl.BlockSpec((NW, BLK, 1), lambda i: (0, i, 0)),
            pl.BlockSpec((130, 60), lambda i: (0, 0)),
        ],
        out_specs=[
            pl.BlockSpec((BLK, 128), lambda i: (i, 0)),
            pl.BlockSpec((BLK, 1), lambda i: (i, 0)),
        ],
        out_shape=[
            jax.ShapeDtypeStruct((NP, 128), f32),
            jax.ShapeDtypeStruct((NP, 1), f32),
        ],
    )(emb, text_p, W_text, b_text, hist, W_gcn1)


def _tc_e_body(acc_ref, y1_ref, dis_ref, b1_ref, w2_ref, y2_ref):
    a = (acc_ref[...] + y1_ref[...])[:, :60]
    dis = dis_ref[...]
    h = jax.nn.relu(dis * a + b1_ref[...])
    z = jnp.dot(h, w2_ref[...], preferred_element_type=f32)
    grow = (pl.program_id(0) * BLK
            + lax.broadcasted_iota(jnp.int32, (BLK, 1), 0))
    y2 = z * dis * (grow != NP - 1).astype(f32)
    y2_ref[...] = jnp.concatenate([y2, jnp.zeros((BLK, 98), f32)], axis=1)


def _tc_e(acc1, y1, dis, b1, W_gcn2):
    return pl.pallas_call(
        _tc_e_body,
        grid=(GRID,),
        in_specs=[
            pl.BlockSpec((BLK, 128), lambda i: (i, 0)),
            pl.BlockSpec((BLK, 128), lambda i: (i, 0)),
            pl.BlockSpec((BLK, 1), lambda i: (i, 0)),
            pl.BlockSpec((1, 60), lambda i: (0, 0)),
            pl.BlockSpec((60, 30), lambda i: (0, 0)),
        ],
        out_specs=pl.BlockSpec((BLK, 128), lambda i: (i, 0)),
        out_shape=jax.ShapeDtypeStruct((NP, 128), f32),
    )(acc1, y1, dis, b1, W_gcn2)


def _tc_g_body(acc_ref, y2_ref, dis_ref, b2_ref, w3_ref, b3_ref, out_ref):
    a = (acc_ref[...] + y2_ref[...])[:, :30]
    h = jax.nn.relu(dis_ref[...] * a + b2_ref[...])
    out_ref[...] = jnp.dot(h, w3_ref[...],
                           preferred_element_type=f32) + b3_ref[...]


def _tc_g(acc2, y2, dis, b2, W_lin3, b3):
    return pl.pallas_call(
        _tc_g_body,
        grid=(GRID,),
        in_specs=[
            pl.BlockSpec((BLK, 128), lambda i: (i, 0)),
            pl.BlockSpec((BLK, 128), lambda i: (i, 0)),
            pl.BlockSpec((BLK, 1), lambda i: (i, 0)),
            pl.BlockSpec((1, 30), lambda i: (0, 0)),
            pl.BlockSpec((30, 18), lambda i: (0, 0)),
            pl.BlockSpec((1, 18), lambda i: (0, 0)),
        ],
        out_specs=pl.BlockSpec((BLK, 18), lambda i: (i, 0)),
        out_shape=jax.ShapeDtypeStruct((NP, 18), f32),
    )(acc2, y2, dis, b2, W_lin3, b3)


def kernel(interest_ids, text_feats, edge_index, emb_table, W_text, b_text,
           W_gcn1, b_gcn1, W_gcn2, b_gcn2, W_lin3, b_lin3):
    i32 = jnp.int32
    fill = jnp.full((EPAD - E,), NP - 1, i32)
    row_p = jnp.concatenate([edge_index[0], fill]).reshape(NS, NCH, ECHUNK)
    col_p = jnp.concatenate([edge_index[1], fill]).reshape(NS, NCH, ECHUNK)
    ids_p = jnp.concatenate(
        [interest_ids.astype(i32), jnp.zeros((IDPAD - N,), i32)]
    ).reshape(NW, IDCH, ECHUNK)
    emb_pad = jnp.pad(emb_table, ((0, 0), (0, 78)))
    text_p = jnp.pad(text_feats, ((0, NP - N), (0, 0)))
    zh = jnp.zeros((NP // 16, 16), f32)
    z128 = jnp.zeros((ASTRIPE, 128), f32)

    hist, emb = _sc_prep(col_p, ids_p, emb_pad, zh)
    hist = hist.reshape(NW, NP, 1)

    y1, dis = _tc_c(emb[:NP], text_p, W_text, b_text.reshape(1, 80), hist,
                    W_gcn1)
    acc1 = _edge_pass(y1, row_p, col_p, z128)
    y2 = _tc_e(acc1, y1, dis, b_gcn1.reshape(1, 60), W_gcn2)
    acc2 = _edge_pass(y2, row_p, col_p, z128)
    out = _tc_g(acc2, y2, dis, b_gcn2.reshape(1, 30), W_lin3,
                b_lin3.reshape(1, 18))
    return out[:N]
